# baseline (speedup 1.0000x reference)
import functools

import jax
import jax.numpy as jnp
from jax import lax
from jax.experimental import pallas as pl
from jax.experimental.pallas import tpu as pltpu
from jax.experimental.pallas import tpu_sc as plsc

D = 256
K = 8192
N = 4608
COMMIT = 0.25

TN = 512
TK = 1024


G = 128
NG = TN // G
NC_CHUNK = TK // 128


def _dist_argmin_body(x_ref, e_hbm, idx_ref, loss_ref,
                      e_ref, esq_ref, rv_ref, rc_ref, xsq_ref, sems):
    i = pl.program_id(0)

    @pl.when(i == 0)
    def _():
        loss_ref[0, 0] = 0.0
        for j in range(K // TK):
            pltpu.make_async_copy(
                e_hbm.at[pl.ds(j * TK, TK)],
                e_ref.at[pl.ds(j * TK, TK)], sems.at[j]).start()

    x = x_ref[...]
    x2 = x + x
    xsq_ref[...] = lax.dot_general(
        x * x, jnp.ones((D, 128), jnp.float32), (((1,), (0,)), ((), ())))

    for j in range(K // TK):
        @pl.when(i == 0)
        def _(j=j):
            pltpu.make_async_copy(
                e_hbm.at[pl.ds(j * TK, TK)],
                e_ref.at[pl.ds(j * TK, TK)], sems.at[j]).wait()
            ecj = e_ref[j * TK:(j + 1) * TK, :]
            esq_ref[:, j * TK:(j + 1) * TK] = lax.dot_general(
                jnp.ones((G, D), jnp.float32), ecj * ecj,
                (((1,), (1,)), ((), ())))

        ec = e_ref[j * TK:(j + 1) * TK, :]
        m2 = lax.dot_general(x2, ec, (((1,), (1,)), ((), ())))
        for g in range(NG):
            r0, r1 = g * G, (g + 1) * G
            xq = xsq_ref[r0:r1, :]
            if j == 0:
                esq_0 = esq_ref[:, 0:128]
                rv = xq - m2[r0:r1, 0:128] + esq_0
                rc = jnp.zeros((G, 128), jnp.int32)
                cols = range(1, NC_CHUNK)
            else:
                rv = rv_ref[r0:r1, :]
                rc = rc_ref[r0:r1, :]
                cols = range(NC_CHUNK)
            for c in cols:
                mc2 = m2[r0:r1, c * 128:(c + 1) * 128]
                esq_c = esq_ref[:, j * TK + c * 128:j * TK + (c + 1) * 128]
                dc = xq - mc2 + esq_c
                take = dc < rv
                rv = jnp.where(take, dc, rv)
                rc = jnp.where(take, j * NC_CHUNK + c, rc)
            rv_ref[r0:r1, :] = rv
            rc_ref[r0:r1, :] = rc

    parts = []
    tot = None
    for g in range(NG):
        r0, r1 = g * G, (g + 1) * G
        rv = rv_ref[r0:r1, :]
        rc = rc_ref[r0:r1, :]
        gmin = jnp.min(rv, axis=1)
        lane = lax.broadcasted_iota(jnp.int32, (G, 128), 1)
        gidx = rc * 128 + lane
        cand = jnp.where(rv == gmin[:, None], gidx, jnp.int32(2**31 - 1))
        parts.append(jnp.min(cand, axis=1))
        s = jnp.sum(gmin)
        tot = s if tot is None else tot + s
    idx_ref[...] = jnp.concatenate(parts)
    loss_ref[0, 0] += tot


def _dist_argmin(x, e):
    return pl.pallas_call(
        _dist_argmin_body,
        grid=(N // TN,),
        in_specs=[
            pl.BlockSpec((TN, D), lambda i: (i, 0)),
            pl.BlockSpec(memory_space=pltpu.HBM),
        ],
        out_specs=[
            pl.BlockSpec((TN,), lambda i: (i,)),
            pl.BlockSpec(memory_space=pltpu.SMEM),
        ],
        out_shape=[
            jax.ShapeDtypeStruct((N,), jnp.int32),
            jax.ShapeDtypeStruct((1, 1), jnp.float32),
        ],
        scratch_shapes=[
            pltpu.VMEM((K, D), jnp.float32),
            pltpu.VMEM((G, K), jnp.float32),
            pltpu.VMEM((TN, 128), jnp.float32),
            pltpu.VMEM((TN, 128), jnp.int32),
            pltpu.VMEM((TN, 128), jnp.float32),
            pltpu.SemaphoreType.DMA((K // TK,)),
        ],
    )(x, e)


_NC = 2
_NS = 16
_NW = _NC * _NS
_BPW = N // _NW
_CH = 72


def _sc_gather_body(table_hbm, idx_hbm, out_hbm,
                    idx0_v, idx1_v, rows0_v, rows1_v, sem):
    wid = lax.axis_index("s") * _NC + lax.axis_index("c")
    base = wid * _BPW
    pltpu.sync_copy(idx_hbm.at[pl.ds(base, _CH)], idx0_v)
    pltpu.sync_copy(idx_hbm.at[pl.ds(base + _CH, _CH)], idx1_v)
    c0 = pltpu.async_copy(table_hbm.at[idx0_v], rows0_v, sem)
    c1 = pltpu.async_copy(table_hbm.at[idx1_v], rows1_v, sem)
    c0.wait()
    c1.wait()
    pltpu.sync_copy(rows0_v, out_hbm.at[pl.ds(base, _CH)])
    pltpu.sync_copy(rows1_v, out_hbm.at[pl.ds(base + _CH, _CH)])


@functools.cache
def _sc_gather_call():
    return pl.kernel(
        _sc_gather_body,
        mesh=plsc.VectorSubcoreMesh(core_axis_name="c", subcore_axis_name="s"),
        out_type=jax.ShapeDtypeStruct((N, D), jnp.float32),
        scratch_types=[
            pltpu.VMEM((_CH,), jnp.int32),
            pltpu.VMEM((_CH,), jnp.int32),
            pltpu.VMEM((_CH, D), jnp.float32),
            pltpu.VMEM((_CH, D), jnp.float32),
            pltpu.SemaphoreType.DMA,
        ],
    )


def _sc_gather(table, idx):
    return _sc_gather_call()(table, idx)


def kernel(inputs, embeddings):
    x = inputs.reshape(-1, D)
    idx, loss_acc = _dist_argmin(x, embeddings)
    q = _sc_gather(embeddings, idx)
    loss = (1.0 + COMMIT) * loss_acc[0, 0] / (N * D)
    return q.reshape(inputs.shape), loss, idx[:, None]


# speedup vs baseline: 1.2269x; 1.2269x over previous
import functools

import jax
import jax.numpy as jnp
from jax import lax
from jax.experimental import pallas as pl
from jax.experimental.pallas import tpu as pltpu
from jax.experimental.pallas import tpu_sc as plsc

D = 256
K = 8192
N = 4608
COMMIT = 0.25

TN = 512
TK = 1024


G = 128
NG = TN // G
NC_CHUNK = TK // 128


def _dist_argmin_body(x_ref, e_hbm, idx_ref, loss_ref,
                      e_ref, esq_ref, rv_ref, rc_ref, xsq_ref, sems):
    i = pl.program_id(0)

    @pl.when(i == 0)
    def _():
        loss_ref[0, 0] = 0.0
        for j in range(K // TK):
            pltpu.make_async_copy(
                e_hbm.at[pl.ds(j * TK, TK)],
                e_ref.at[pl.ds(j * TK, TK)], sems.at[j]).start()
        ones_g = jnp.ones((G, D), jnp.float32)
        for j in range(K // TK):
            pltpu.make_async_copy(
                e_hbm.at[pl.ds(j * TK, TK)],
                e_ref.at[pl.ds(j * TK, TK)], sems.at[j]).wait()
            ecj = e_ref[j * TK:(j + 1) * TK, :]
            esq_ref[:, j * TK:(j + 1) * TK] = lax.dot_general(
                ones_g, ecj * ecj, (((1,), (1,)), ((), ())))

    x = x_ref[...]
    x2 = x + x
    xsq_ref[...] = lax.dot_general(
        x * x, jnp.ones((D, 128), jnp.float32), (((1,), (0,)), ((), ())))

    for j in range(K // TK):
        ec = e_ref[j * TK:(j + 1) * TK, :]
        m2 = lax.dot_general(x2, ec, (((1,), (1,)), ((), ())))
        for g in range(NG):
            r0, r1 = g * G, (g + 1) * G
            xq = xsq_ref[r0:r1, :]
            if j == 0:
                esq_0 = esq_ref[:, 0:128]
                rv = xq - m2[r0:r1, 0:128] + esq_0
                rc = jnp.zeros((G, 128), jnp.int32)
                cols = range(1, NC_CHUNK)
            else:
                rv = rv_ref[r0:r1, :]
                rc = rc_ref[r0:r1, :]
                cols = range(NC_CHUNK)
            for c in cols:
                mc2 = m2[r0:r1, c * 128:(c + 1) * 128]
                esq_c = esq_ref[:, j * TK + c * 128:j * TK + (c + 1) * 128]
                dc = xq - mc2 + esq_c
                take = dc < rv
                rv = jnp.where(take, dc, rv)
                rc = jnp.where(take, j * NC_CHUNK + c, rc)
            rv_ref[r0:r1, :] = rv
            rc_ref[r0:r1, :] = rc

    parts = []
    tot = None
    for g in range(NG):
        r0, r1 = g * G, (g + 1) * G
        rv = rv_ref[r0:r1, :]
        rc = rc_ref[r0:r1, :]
        gmin = jnp.min(rv, axis=1)
        lane = lax.broadcasted_iota(jnp.int32, (G, 128), 1)
        gidx = rc * 128 + lane
        cand = jnp.where(rv == gmin[:, None], gidx, jnp.int32(2**31 - 1))
        parts.append(jnp.min(cand, axis=1))
        s = jnp.sum(gmin)
        tot = s if tot is None else tot + s
    idx_ref[...] = jnp.concatenate(parts)
    loss_ref[0, 0] += tot


def _dist_argmin(x, e):
    return pl.pallas_call(
        _dist_argmin_body,
        grid=(N // TN,),
        in_specs=[
            pl.BlockSpec((TN, D), lambda i: (i, 0)),
            pl.BlockSpec(memory_space=pltpu.HBM),
        ],
        out_specs=[
            pl.BlockSpec((TN,), lambda i: (i,)),
            pl.BlockSpec(memory_space=pltpu.SMEM),
        ],
        out_shape=[
            jax.ShapeDtypeStruct((N,), jnp.int32),
            jax.ShapeDtypeStruct((1, 1), jnp.float32),
        ],
        scratch_shapes=[
            pltpu.VMEM((K, D), jnp.float32),
            pltpu.VMEM((G, K), jnp.float32),
            pltpu.VMEM((TN, 128), jnp.float32),
            pltpu.VMEM((TN, 128), jnp.int32),
            pltpu.VMEM((TN, 128), jnp.float32),
            pltpu.SemaphoreType.DMA((K // TK,)),
        ],
    )(x, e)


_NC = 2
_NS = 16
_NW = _NC * _NS
_BPW = N // _NW
_CH = 72


def _sc_gather_body(table_hbm, idx_hbm, out_hbm,
                    idx0_v, idx1_v, rows0_v, rows1_v, sem):
    wid = lax.axis_index("s") * _NC + lax.axis_index("c")
    base = wid * _BPW
    pltpu.sync_copy(idx_hbm.at[pl.ds(base, _CH)], idx0_v)
    pltpu.sync_copy(idx_hbm.at[pl.ds(base + _CH, _CH)], idx1_v)
    c0 = pltpu.async_copy(table_hbm.at[idx0_v], rows0_v, sem)
    c1 = pltpu.async_copy(table_hbm.at[idx1_v], rows1_v, sem)
    c0.wait()
    c1.wait()
    pltpu.sync_copy(rows0_v, out_hbm.at[pl.ds(base, _CH)])
    pltpu.sync_copy(rows1_v, out_hbm.at[pl.ds(base + _CH, _CH)])


@functools.cache
def _sc_gather_call():
    return pl.kernel(
        _sc_gather_body,
        mesh=plsc.VectorSubcoreMesh(core_axis_name="c", subcore_axis_name="s"),
        out_type=jax.ShapeDtypeStruct((N, D), jnp.float32),
        scratch_types=[
            pltpu.VMEM((_CH,), jnp.int32),
            pltpu.VMEM((_CH,), jnp.int32),
            pltpu.VMEM((_CH, D), jnp.float32),
            pltpu.VMEM((_CH, D), jnp.float32),
            pltpu.SemaphoreType.DMA,
        ],
    )


def _sc_gather(table, idx):
    return _sc_gather_call()(table, idx)


def kernel(inputs, embeddings):
    x = inputs.reshape(-1, D)
    idx, loss_acc = _dist_argmin(x, embeddings)
    q = _sc_gather(embeddings, idx)
    loss = (1.0 + COMMIT) * loss_acc[0, 0] / (N * D)
    return q.reshape(inputs.shape), loss, idx[:, None]


# speedup vs baseline: 1.2286x; 1.0014x over previous
import functools

import jax
import jax.numpy as jnp
from jax import lax
from jax.experimental import pallas as pl
from jax.experimental.pallas import tpu as pltpu
from jax.experimental.pallas import tpu_sc as plsc

D = 256
K = 8192
N = 4608
COMMIT = 0.25

TN = 512
TK = 2048


G = 128
NG = TN // G
NC_CHUNK = TK // 128


def _dist_argmin_body(x_ref, e_hbm, idx_ref, loss_ref,
                      e_ref, esq_ref, rv_ref, rc_ref, xsq_ref, sems):
    i = pl.program_id(0)

    @pl.when(i == 0)
    def _():
        loss_ref[0, 0] = 0.0
        for j in range(K // TK):
            pltpu.make_async_copy(
                e_hbm.at[pl.ds(j * TK, TK)],
                e_ref.at[pl.ds(j * TK, TK)], sems.at[j]).start()
        ones_g = jnp.ones((G, D), jnp.float32)
        for j in range(K // TK):
            pltpu.make_async_copy(
                e_hbm.at[pl.ds(j * TK, TK)],
                e_ref.at[pl.ds(j * TK, TK)], sems.at[j]).wait()
            ecj = e_ref[j * TK:(j + 1) * TK, :]
            esq_ref[:, j * TK:(j + 1) * TK] = lax.dot_general(
                ones_g, ecj * ecj, (((1,), (1,)), ((), ())))

    x = x_ref[...]
    x2 = x + x
    xsq_ref[...] = lax.dot_general(
        x * x, jnp.ones((D, 128), jnp.float32), (((1,), (0,)), ((), ())))

    for j in range(K // TK):
        ec = e_ref[j * TK:(j + 1) * TK, :]
        m2 = lax.dot_general(x2, ec, (((1,), (1,)), ((), ())))
        for g in range(NG):
            r0, r1 = g * G, (g + 1) * G
            xq = xsq_ref[r0:r1, :]
            if j == 0:
                esq_0 = esq_ref[:, 0:128]
                rv = xq - m2[r0:r1, 0:128] + esq_0
                rc = jnp.zeros((G, 128), jnp.int32)
                cols = range(1, NC_CHUNK)
            else:
                rv = rv_ref[r0:r1, :]
                rc = rc_ref[r0:r1, :]
                cols = range(NC_CHUNK)
            for c in cols:
                mc2 = m2[r0:r1, c * 128:(c + 1) * 128]
                esq_c = esq_ref[:, j * TK + c * 128:j * TK + (c + 1) * 128]
                dc = xq - mc2 + esq_c
                take = dc < rv
                rv = jnp.where(take, dc, rv)
                rc = jnp.where(take, j * NC_CHUNK + c, rc)
            rv_ref[r0:r1, :] = rv
            rc_ref[r0:r1, :] = rc

    parts = []
    tot = None
    for g in range(NG):
        r0, r1 = g * G, (g + 1) * G
        rv = rv_ref[r0:r1, :]
        rc = rc_ref[r0:r1, :]
        gmin = jnp.min(rv, axis=1)
        lane = lax.broadcasted_iota(jnp.int32, (G, 128), 1)
        gidx = rc * 128 + lane
        cand = jnp.where(rv == gmin[:, None], gidx, jnp.int32(2**31 - 1))
        parts.append(jnp.min(cand, axis=1))
        s = jnp.sum(gmin)
        tot = s if tot is None else tot + s
    idx_ref[...] = jnp.concatenate(parts)
    loss_ref[0, 0] += tot


def _dist_argmin(x, e):
    return pl.pallas_call(
        _dist_argmin_body,
        grid=(N // TN,),
        in_specs=[
            pl.BlockSpec((TN, D), lambda i: (i, 0)),
            pl.BlockSpec(memory_space=pltpu.HBM),
        ],
        out_specs=[
            pl.BlockSpec((TN,), lambda i: (i,)),
            pl.BlockSpec(memory_space=pltpu.SMEM),
        ],
        out_shape=[
            jax.ShapeDtypeStruct((N,), jnp.int32),
            jax.ShapeDtypeStruct((1, 1), jnp.float32),
        ],
        scratch_shapes=[
            pltpu.VMEM((K, D), jnp.float32),
            pltpu.VMEM((G, K), jnp.float32),
            pltpu.VMEM((TN, 128), jnp.float32),
            pltpu.VMEM((TN, 128), jnp.int32),
            pltpu.VMEM((TN, 128), jnp.float32),
            pltpu.SemaphoreType.DMA((K // TK,)),
        ],
    )(x, e)


_NC = 2
_NS = 16
_NW = _NC * _NS
_BPW = N // _NW
_CH = 72


def _sc_gather_body(table_hbm, idx_hbm, out_hbm,
                    idx0_v, idx1_v, rows0_v, rows1_v, sem):
    wid = lax.axis_index("s") * _NC + lax.axis_index("c")
    base = wid * _BPW
    pltpu.sync_copy(idx_hbm.at[pl.ds(base, _CH)], idx0_v)
    pltpu.sync_copy(idx_hbm.at[pl.ds(base + _CH, _CH)], idx1_v)
    c0 = pltpu.async_copy(table_hbm.at[idx0_v], rows0_v, sem)
    c1 = pltpu.async_copy(table_hbm.at[idx1_v], rows1_v, sem)
    c0.wait()
    c1.wait()
    pltpu.sync_copy(rows0_v, out_hbm.at[pl.ds(base, _CH)])
    pltpu.sync_copy(rows1_v, out_hbm.at[pl.ds(base + _CH, _CH)])


@functools.cache
def _sc_gather_call():
    return pl.kernel(
        _sc_gather_body,
        mesh=plsc.VectorSubcoreMesh(core_axis_name="c", subcore_axis_name="s"),
        out_type=jax.ShapeDtypeStruct((N, D), jnp.float32),
        scratch_types=[
            pltpu.VMEM((_CH,), jnp.int32),
            pltpu.VMEM((_CH,), jnp.int32),
            pltpu.VMEM((_CH, D), jnp.float32),
            pltpu.VMEM((_CH, D), jnp.float32),
            pltpu.SemaphoreType.DMA,
        ],
    )


def _sc_gather(table, idx):
    return _sc_gather_call()(table, idx)


def kernel(inputs, embeddings):
    x = inputs.reshape(-1, D)
    idx, loss_acc = _dist_argmin(x, embeddings)
    q = _sc_gather(embeddings, idx)
    loss = (1.0 + COMMIT) * loss_acc[0, 0] / (N * D)
    return q.reshape(inputs.shape), loss, idx[:, None]


# speedup vs baseline: 1.2345x; 1.0048x over previous
import functools

import jax
import jax.numpy as jnp
from jax import lax
from jax.experimental import pallas as pl
from jax.experimental.pallas import tpu as pltpu
from jax.experimental.pallas import tpu_sc as plsc

D = 256
K = 8192
N = 4608
COMMIT = 0.25

TN = 1536
TK = 2048


G = 128
NG = TN // G
NC_CHUNK = TK // 128


def _dist_argmin_body(x_ref, e_hbm, idx_ref, loss_ref,
                      e_ref, esq_ref, rv_ref, rc_ref, xsq_ref, sems):
    i = pl.program_id(0)

    @pl.when(i == 0)
    def _():
        loss_ref[0, 0] = 0.0
        for j in range(K // TK):
            pltpu.make_async_copy(
                e_hbm.at[pl.ds(j * TK, TK)],
                e_ref.at[pl.ds(j * TK, TK)], sems.at[j]).start()
        ones_g = jnp.ones((G, D), jnp.float32)
        for j in range(K // TK):
            pltpu.make_async_copy(
                e_hbm.at[pl.ds(j * TK, TK)],
                e_ref.at[pl.ds(j * TK, TK)], sems.at[j]).wait()
            ecj = e_ref[j * TK:(j + 1) * TK, :]
            esq_ref[:, j * TK:(j + 1) * TK] = lax.dot_general(
                ones_g, ecj * ecj, (((1,), (1,)), ((), ())))

    x = x_ref[...]
    x2 = x + x
    xsq_ref[...] = lax.dot_general(
        x * x, jnp.ones((D, 128), jnp.float32), (((1,), (0,)), ((), ())))

    for j in range(K // TK):
        ec = e_ref[j * TK:(j + 1) * TK, :]
        m2 = lax.dot_general(x2, ec, (((1,), (1,)), ((), ())))
        for g in range(NG):
            r0, r1 = g * G, (g + 1) * G
            xq = xsq_ref[r0:r1, :]
            if j == 0:
                esq_0 = esq_ref[:, 0:128]
                rv = xq - m2[r0:r1, 0:128] + esq_0
                rc = jnp.zeros((G, 128), jnp.int32)
                cols = range(1, NC_CHUNK)
            else:
                rv = rv_ref[r0:r1, :]
                rc = rc_ref[r0:r1, :]
                cols = range(NC_CHUNK)
            for c in cols:
                mc2 = m2[r0:r1, c * 128:(c + 1) * 128]
                esq_c = esq_ref[:, j * TK + c * 128:j * TK + (c + 1) * 128]
                dc = xq - mc2 + esq_c
                take = dc < rv
                rv = jnp.where(take, dc, rv)
                rc = jnp.where(take, j * NC_CHUNK + c, rc)
            rv_ref[r0:r1, :] = rv
            rc_ref[r0:r1, :] = rc

    parts = []
    tot = None
    for g in range(NG):
        r0, r1 = g * G, (g + 1) * G
        rv = rv_ref[r0:r1, :]
        rc = rc_ref[r0:r1, :]
        gmin = jnp.min(rv, axis=1)
        lane = lax.broadcasted_iota(jnp.int32, (G, 128), 1)
        gidx = rc * 128 + lane
        cand = jnp.where(rv == gmin[:, None], gidx, jnp.int32(2**31 - 1))
        parts.append(jnp.min(cand, axis=1))
        s = jnp.sum(gmin)
        tot = s if tot is None else tot + s
    idx_ref[...] = jnp.concatenate([p[None, :] for p in parts])[None]
    loss_ref[0, 0] += tot


def _dist_argmin(x, e):
    return pl.pallas_call(
        _dist_argmin_body,
        grid=(N // TN,),
        in_specs=[
            pl.BlockSpec((TN, D), lambda i: (i, 0)),
            pl.BlockSpec(memory_space=pltpu.HBM),
        ],
        out_specs=[
            pl.BlockSpec((1, NG, G), lambda i: (i, 0, 0)),
            pl.BlockSpec(memory_space=pltpu.SMEM),
        ],
        out_shape=[
            jax.ShapeDtypeStruct((N // TN, NG, G), jnp.int32),
            jax.ShapeDtypeStruct((1, 1), jnp.float32),
        ],
        scratch_shapes=[
            pltpu.VMEM((K, D), jnp.float32),
            pltpu.VMEM((G, K), jnp.float32),
            pltpu.VMEM((TN, 128), jnp.float32),
            pltpu.VMEM((TN, 128), jnp.int32),
            pltpu.VMEM((TN, 128), jnp.float32),
            pltpu.SemaphoreType.DMA((K // TK,)),
        ],
    )(x, e)


_NC = 2
_NS = 16
_NW = _NC * _NS
_BPW = N // _NW
_CH = 72


def _sc_gather_body(table_hbm, idx_hbm, out_hbm,
                    idx0_v, idx1_v, rows0_v, rows1_v, sem):
    wid = lax.axis_index("s") * _NC + lax.axis_index("c")
    base = wid * _BPW
    pltpu.sync_copy(idx_hbm.at[pl.ds(base, _CH)], idx0_v)
    pltpu.sync_copy(idx_hbm.at[pl.ds(base + _CH, _CH)], idx1_v)
    c0 = pltpu.async_copy(table_hbm.at[idx0_v], rows0_v, sem)
    c1 = pltpu.async_copy(table_hbm.at[idx1_v], rows1_v, sem)
    c0.wait()
    c1.wait()
    pltpu.sync_copy(rows0_v, out_hbm.at[pl.ds(base, _CH)])
    pltpu.sync_copy(rows1_v, out_hbm.at[pl.ds(base + _CH, _CH)])


@functools.cache
def _sc_gather_call():
    return pl.kernel(
        _sc_gather_body,
        mesh=plsc.VectorSubcoreMesh(core_axis_name="c", subcore_axis_name="s"),
        out_type=jax.ShapeDtypeStruct((N, D), jnp.float32),
        scratch_types=[
            pltpu.VMEM((_CH,), jnp.int32),
            pltpu.VMEM((_CH,), jnp.int32),
            pltpu.VMEM((_CH, D), jnp.float32),
            pltpu.VMEM((_CH, D), jnp.float32),
            pltpu.SemaphoreType.DMA,
        ],
    )


def _sc_gather(table, idx):
    return _sc_gather_call()(table, idx)


def kernel(inputs, embeddings):
    x = inputs.reshape(-1, D)
    idx3, loss_acc = _dist_argmin(x, embeddings)
    idx = idx3.reshape(N)
    q = _sc_gather(embeddings, idx)
    loss = (1.0 + COMMIT) * loss_acc[0, 0] / (N * D)
    return q.reshape(inputs.shape), loss, idx[:, None]
